# Initial kernel scaffold; baseline (speedup 1.0000x reference)
import jax
import jax.numpy as jnp
from jax.experimental import pallas as pl


def kernel(x, edge_index, Wl0, bl0, Wr0, Wl1, bl1, Wr1, Wlo, blo, Wro):
    raise NotImplementedError("write your pallas kernel here")


# speedup vs baseline: 5.6698x; 5.6698x over previous
import functools

import jax
import jax.numpy as jnp
from jax import lax
from jax.experimental import pallas as pl
from jax.experimental.pallas import tpu as pltpu
from jax.experimental.pallas import tpu_sc as plsc

N = 10000
N_PAD = 10240
D = 128
NC = 2
NS = 16
NW = NC * NS
CHUNK = 64
ROWS_PT = N_PAD // NS
PAD_DST = N


def _make_agg(feat_width, with_deg, ept):
    nchunk = ept // CHUNK
    mesh = plsc.VectorSubcoreMesh(core_axis_name="c", subcore_axis_name="s")
    tbl_shape = (N_PAD, feat_width) if feat_width > 1 else (N_PAD,)
    buf_shape = (CHUNK, feat_width) if feat_width > 1 else (CHUNK,)

    out_type = [jax.ShapeDtypeStruct((NC,) + tbl_shape, jnp.float32)]
    if with_deg:
        out_type.append(jax.ShapeDtypeStruct((NC, N_PAD), jnp.float32))

    scratch_types = [
        pltpu.VMEM((ept,), jnp.int32),
        pltpu.VMEM((nchunk, CHUNK), jnp.int32),
        pltpu.VMEM(buf_shape, jnp.float32),
        pltpu.VMEM(buf_shape, jnp.float32),
        pltpu.VMEM_SHARED(tbl_shape, jnp.float32),
        pltpu.SemaphoreType.DMA,
        pltpu.SemaphoreType.DMA,
    ]
    if with_deg:
        scratch_types += [
            pltpu.VMEM((CHUNK,), jnp.float32),
            pltpu.VMEM((CHUNK,), jnp.float32),
            pltpu.VMEM_SHARED((N_PAD,), jnp.float32),
        ]

    @functools.partial(pl.kernel, mesh=mesh, out_type=out_type,
                       scratch_types=scratch_types)
    def k(table, src, dst, *refs):
        if with_deg:
            (out, deg_out, src_v, dst_v, buf0, buf1, acc,
             sem0, sem1, ones_v, z1buf, dacc) = refs
        else:
            (out, src_v, dst_v, buf0, buf1, acc, sem0, sem1) = refs
        c = lax.axis_index("c")
        s = lax.axis_index("s")
        wid = c * NS + s

        pltpu.sync_copy(src.at[wid], src_v)
        pltpu.sync_copy(dst.at[wid], dst_v)

        zvec = jnp.zeros((16,), jnp.float32)
        if feat_width > 1:
            def zrow(r, carry):
                for k16 in range(feat_width // 16):
                    buf0[r, pl.ds(16 * k16, 16)] = zvec
                return carry
            lax.fori_loop(0, CHUNK, zrow, 0)
        else:
            def zrow(r, carry):
                buf0[pl.ds(16 * r, 16)] = zvec
                return carry
            lax.fori_loop(0, CHUNK // 16, zrow, 0)
        if with_deg:
            def orow(r, carry):
                ones_v[pl.ds(16 * r, 16)] = jnp.ones((16,), jnp.float32)
                z1buf[pl.ds(16 * r, 16)] = zvec
                return carry
            lax.fori_loop(0, CHUNK // 16, orow, 0)

        for t in range(ROWS_PT // CHUNK):
            row = s * ROWS_PT + t * CHUNK
            pltpu.sync_copy(buf0, acc.at[pl.ds(row, CHUNK)])
        if with_deg:
            for t in range(ROWS_PT // CHUNK):
                row = s * ROWS_PT + t * CHUNK
                pltpu.sync_copy(z1buf, dacc.at[pl.ds(row, CHUNK)])
        plsc.subcore_barrier()

        def body(i, carry):
            j0 = 2 * i
            j1 = 2 * i + 1
            g0 = pltpu.async_copy(
                table.at[src_v.at[pl.ds(j0 * CHUNK, CHUNK)]], buf0, sem0)
            g1 = pltpu.async_copy(
                table.at[src_v.at[pl.ds(j1 * CHUNK, CHUNK)]], buf1, sem1)
            g0.wait()
            pltpu.sync_copy(buf0, acc.at[dst_v.at[j0]], add=True)
            g1.wait()
            pltpu.sync_copy(buf1, acc.at[dst_v.at[j1]], add=True)
            if with_deg:
                pltpu.sync_copy(ones_v, dacc.at[dst_v.at[j0]], add=True)
                pltpu.sync_copy(ones_v, dacc.at[dst_v.at[j1]], add=True)
            return carry
        lax.fori_loop(0, nchunk // 2, body, 0)

        plsc.subcore_barrier()

        if feat_width > 1:
            for t in range(ROWS_PT // CHUNK):
                row = s * ROWS_PT + t * CHUNK
                pltpu.sync_copy(acc.at[pl.ds(row, CHUNK)],
                                out.at[c, pl.ds(row, CHUNK)])
        else:
            pltpu.sync_copy(acc.at[pl.ds(s * ROWS_PT, ROWS_PT)],
                            out.at[c, pl.ds(s * ROWS_PT, ROWS_PT)])
        if with_deg:
            pltpu.sync_copy(dacc.at[pl.ds(s * ROWS_PT, ROWS_PT)],
                            deg_out.at[c, pl.ds(s * ROWS_PT, ROWS_PT)])

    return k


BM = 256


def _c0_body(p_ref, x_ref, d_ref, wl_ref, bl_ref, wr_ref, h_ref, rdeg_ref):
    d = jnp.maximum(d_ref[0] + d_ref[1], 1.0)
    rd = 1.0 / d
    mean = (p_ref[0] + p_ref[1]) * rd
    h = (jnp.dot(mean, wl_ref[...], preferred_element_type=jnp.float32)
         + bl_ref[...]
         + jnp.dot(x_ref[...], wr_ref[...], preferred_element_type=jnp.float32))
    h_ref[...] = jnp.maximum(h, 0.0)
    rdeg_ref[...] = rd


def _c1_body(q_ref, h_ref, rdeg_ref, wl_ref, bl_ref, wr_ref, wz_ref, bz_ref,
             z_ref):
    mean = (q_ref[0] + q_ref[1]) * rdeg_ref[...]
    h2 = (jnp.dot(mean, wl_ref[...], preferred_element_type=jnp.float32)
          + bl_ref[...]
          + jnp.dot(h_ref[...], wr_ref[...], preferred_element_type=jnp.float32))
    h2 = jnp.maximum(h2, 0.0)
    z_ref[...] = jnp.dot(h2, wz_ref[...], preferred_element_type=jnp.float32) \
        + bz_ref[...]


def _final_body(r_ref, rdeg_ref, zrb_ref, o_ref):
    o_ref[...] = jax.nn.sigmoid((r_ref[0] + r_ref[1]) * rdeg_ref[...]
                                + zrb_ref[...])


def _w_spec():
    return pl.BlockSpec((D, D), lambda i: (0, 0))


def _b_spec():
    return pl.BlockSpec((1, D), lambda i: (0, 0))


def _col_spec():
    return pl.BlockSpec((BM, 1), lambda i: (i, 0))


def _row_spec():
    return pl.BlockSpec((BM, D), lambda i: (i, 0))


def _p_spec():
    return pl.BlockSpec((NC, BM, D), lambda i: (0, i, 0))


def _d_spec():
    return pl.BlockSpec((NC, BM, 1), lambda i: (0, i, 0))


def kernel(x, edge_index, Wl0, bl0, Wr0, Wl1, bl1, Wr1, Wlo, blo, Wro):
    n, d_feat = x.shape
    e = edge_index.shape[1]
    nchunk = -(-e // (NW * CHUNK))
    nchunk += nchunk % 2
    ept = nchunk * CHUNK
    e_pad = ept * NW

    ei = edge_index.astype(jnp.int32)
    src = jnp.concatenate(
        [ei[0], jnp.zeros((e_pad - e,), jnp.int32)]).reshape(NW, ept)
    dst = jnp.concatenate(
        [ei[1], jnp.full((e_pad - e,), PAD_DST, jnp.int32)]
    ).reshape(NW, ept // CHUNK, CHUNK)
    xp = jnp.pad(x, ((0, N_PAD - n), (0, 0)))

    grid = (N_PAD // BM,)

    agg0 = _make_agg(D, True, ept)
    p, deg = agg0(xp, src, dst)
    deg3 = deg[:, :, None]

    h1, rdeg = pl.pallas_call(
        _c0_body,
        grid=grid,
        in_specs=[_p_spec(), _row_spec(), _d_spec(),
                  _w_spec(), _b_spec(), _w_spec()],
        out_specs=[_row_spec(), _col_spec()],
        out_shape=[jax.ShapeDtypeStruct((N_PAD, D), jnp.float32),
                   jax.ShapeDtypeStruct((N_PAD, 1), jnp.float32)],
    )(p, xp, deg3, Wl0, bl0.reshape(1, D), Wr0)

    agg1 = _make_agg(D, False, ept)
    (q,) = agg1(h1, src, dst)

    wz = jnp.concatenate(
        [Wlo, Wro, jnp.zeros((D, D - 2 * Wlo.shape[1]), jnp.float32)], axis=1)
    bz = jnp.zeros((1, D), jnp.float32).at[0, 1].set(blo[0])
    z = pl.pallas_call(
        _c1_body,
        grid=grid,
        in_specs=[_p_spec(), _row_spec(), _col_spec(),
                  _w_spec(), _b_spec(), _w_spec(), _w_spec(), _b_spec()],
        out_specs=_row_spec(),
        out_shape=jax.ShapeDtypeStruct((N_PAD, D), jnp.float32),
    )(q, h1, rdeg, Wl1, bl1.reshape(1, D), Wr1, wz, bz)

    zl = z[:, 0]
    zrb = z[:, 1:2]
    agg2 = _make_agg(1, False, ept)
    (r,) = agg2(zl, src, dst)

    out = pl.pallas_call(
        _final_body,
        grid=grid,
        in_specs=[_d_spec(), _col_spec(), _col_spec()],
        out_specs=_col_spec(),
        out_shape=jax.ShapeDtypeStruct((N_PAD, 1), jnp.float32),
    )(r[:, :, None], rdeg, zrb)

    return out[:n]


# speedup vs baseline: 7.0059x; 1.2357x over previous
import functools

import jax
import jax.numpy as jnp
from jax import lax
from jax.experimental import pallas as pl
from jax.experimental.pallas import tpu as pltpu
from jax.experimental.pallas import tpu_sc as plsc

N = 10000
N_PAD = 10240
D = 128
NC = 2
NS = 16
NW = NC * NS
CHUNK = 64
ZC = 64
ROWS_PT = N_PAD // NS
PAD_DST = N


def _make_agg(feat_width, with_deg, ept):
    nchunk = ept // CHUNK
    mesh = plsc.VectorSubcoreMesh(core_axis_name="c", subcore_axis_name="s")
    tbl_shape = (N_PAD, feat_width) if feat_width > 1 else (N_PAD,)
    buf_shape = (CHUNK, feat_width) if feat_width > 1 else (CHUNK,)

    out_type = [jax.ShapeDtypeStruct((NC,) + tbl_shape, jnp.float32)]
    if with_deg:
        out_type.append(jax.ShapeDtypeStruct((NC, N_PAD), jnp.float32))

    scratch_types = [
        pltpu.VMEM((ept,), jnp.int32),
        pltpu.VMEM((nchunk, CHUNK), jnp.int32),
        pltpu.VMEM(buf_shape, jnp.float32),
        pltpu.VMEM(buf_shape, jnp.float32),
        pltpu.VMEM_SHARED(tbl_shape, jnp.float32),
        pltpu.SemaphoreType.DMA,
        pltpu.SemaphoreType.DMA,
    ]
    if with_deg:
        scratch_types += [
            pltpu.VMEM((CHUNK,), jnp.float32),
            pltpu.VMEM((CHUNK,), jnp.float32),
            pltpu.VMEM_SHARED((N_PAD,), jnp.float32),
        ]

    @functools.partial(pl.kernel, mesh=mesh, out_type=out_type,
                       scratch_types=scratch_types)
    def k(table, src, dst, *refs):
        if with_deg:
            (out, deg_out, src_v, dst_v, buf0, buf1, acc,
             sem0, sem1, ones_v, z1buf, dacc) = refs
        else:
            (out, src_v, dst_v, buf0, buf1, acc, sem0, sem1) = refs
        c = lax.axis_index("c")
        s = lax.axis_index("s")
        wid = c * NS + s

        pltpu.sync_copy(src.at[wid], src_v)
        pltpu.sync_copy(dst.at[wid], dst_v)

        zvec = jnp.zeros((16,), jnp.float32)
        if feat_width > 1:
            def zrow(r, carry):
                for k16 in range(feat_width // 16):
                    buf0[r, pl.ds(16 * k16, 16)] = zvec
                return carry
            lax.fori_loop(0, ZC, zrow, 0)
        else:
            def zrow(r, carry):
                buf0[pl.ds(16 * r, 16)] = zvec
                return carry
            lax.fori_loop(0, ZC // 16, zrow, 0)
        if with_deg:
            def orow(r, carry):
                ones_v[pl.ds(16 * r, 16)] = jnp.ones((16,), jnp.float32)
                return carry
            lax.fori_loop(0, CHUNK // 16, orow, 0)
            def z1row(r, carry):
                z1buf[pl.ds(16 * r, 16)] = zvec
                return carry
            lax.fori_loop(0, ZC // 16, z1row, 0)

        zsrc = buf0.at[pl.ds(0, ZC)]
        for t in range(ROWS_PT // ZC):
            row = s * ROWS_PT + t * ZC
            pltpu.sync_copy(zsrc, acc.at[pl.ds(row, ZC)])
        if with_deg:
            for t in range(ROWS_PT // ZC):
                row = s * ROWS_PT + t * ZC
                pltpu.sync_copy(z1buf.at[pl.ds(0, ZC)], dacc.at[pl.ds(row, ZC)])
        plsc.subcore_barrier()

        def start_gather(j, buf, sem):
            return pltpu.async_copy(
                table.at[src_v.at[pl.ds(j * CHUNK, CHUNK)]], buf, sem)

        def wait_gather(j, buf, sem):
            pltpu.make_async_copy(
                table.at[src_v.at[pl.ds(j * CHUNK, CHUNK)]], buf, sem).wait()

        def scatter(j, buf):
            pltpu.sync_copy(buf, acc.at[dst_v.at[j]], add=True)
            if with_deg:
                pltpu.sync_copy(ones_v, dacc.at[dst_v.at[j]], add=True)

        npair = nchunk // 2
        start_gather(0, buf0, sem0)

        def body(i, carry):
            j0 = 2 * i
            start_gather(j0 + 1, buf1, sem1)
            wait_gather(j0, buf0, sem0)
            scatter(j0, buf0)

            @pl.when(i < npair - 1)
            def _():
                start_gather(j0 + 2, buf0, sem0)
            wait_gather(j0 + 1, buf1, sem1)
            scatter(j0 + 1, buf1)
            return carry
        lax.fori_loop(0, npair, body, 0)

        plsc.subcore_barrier()

        if feat_width > 1:
            for t in range(ROWS_PT // CHUNK):
                row = s * ROWS_PT + t * CHUNK
                pltpu.sync_copy(acc.at[pl.ds(row, CHUNK)],
                                out.at[c, pl.ds(row, CHUNK)])
        else:
            pltpu.sync_copy(acc.at[pl.ds(s * ROWS_PT, ROWS_PT)],
                            out.at[c, pl.ds(s * ROWS_PT, ROWS_PT)])
        if with_deg:
            pltpu.sync_copy(dacc.at[pl.ds(s * ROWS_PT, ROWS_PT)],
                            deg_out.at[c, pl.ds(s * ROWS_PT, ROWS_PT)])

    return k


BM = 200


def _c0_body(p_ref, x_ref, d_ref, wl_ref, bl_ref, wr_ref, h_ref, rdeg_ref):
    d = jnp.maximum(d_ref[0] + d_ref[1], 1.0)
    rd = 1.0 / d
    mean = (p_ref[0] + p_ref[1]) * rd
    h = (jnp.dot(mean, wl_ref[...], preferred_element_type=jnp.float32)
         + bl_ref[...]
         + jnp.dot(x_ref[...], wr_ref[...], preferred_element_type=jnp.float32))
    h_ref[...] = jnp.maximum(h, 0.0)
    rdeg_ref[...] = rd


def _c1_body(q_ref, h_ref, rdeg_ref, wl_ref, bl_ref, wr_ref, wz_ref, bz_ref,
             z_ref):
    mean = (q_ref[0] + q_ref[1]) * rdeg_ref[...]
    h2 = (jnp.dot(mean, wl_ref[...], preferred_element_type=jnp.float32)
          + bl_ref[...]
          + jnp.dot(h_ref[...], wr_ref[...], preferred_element_type=jnp.float32))
    h2 = jnp.maximum(h2, 0.0)
    z_ref[...] = jnp.dot(h2, wz_ref[...], preferred_element_type=jnp.float32) \
        + bz_ref[...]


def _final_body(r_ref, rdeg_ref, zrb_ref, o_ref):
    o_ref[...] = jax.nn.sigmoid((r_ref[0] + r_ref[1]) * rdeg_ref[...]
                                + zrb_ref[...])


def _w_spec():
    return pl.BlockSpec((D, D), lambda i: (0, 0))


def _b_spec():
    return pl.BlockSpec((1, D), lambda i: (0, 0))


def _col_spec():
    return pl.BlockSpec((BM, 1), lambda i: (i, 0))


def _row_spec():
    return pl.BlockSpec((BM, D), lambda i: (i, 0))


def _p_spec():
    return pl.BlockSpec((NC, BM, D), lambda i: (0, i, 0))


def _d_spec():
    return pl.BlockSpec((NC, BM, 1), lambda i: (0, i, 0))


def kernel(x, edge_index, Wl0, bl0, Wr0, Wl1, bl1, Wr1, Wlo, blo, Wro):
    n, d_feat = x.shape
    e = edge_index.shape[1]
    nchunk = -(-e // (NW * CHUNK))
    nchunk += nchunk % 2
    ept = nchunk * CHUNK
    e_pad = ept * NW

    ei = edge_index.astype(jnp.int32)
    src = jnp.concatenate(
        [ei[0], jnp.zeros((e_pad - e,), jnp.int32)]).reshape(NW, ept)
    dst = jnp.concatenate(
        [ei[1], jnp.full((e_pad - e,), PAD_DST, jnp.int32)]
    ).reshape(NW, ept // CHUNK, CHUNK)

    grid = (n // BM,)

    agg0 = _make_agg(D, True, ept)
    p, deg = agg0(x, src, dst)
    deg3 = deg[:, :, None]

    h1, rdeg = pl.pallas_call(
        _c0_body,
        grid=grid,
        in_specs=[_p_spec(), _row_spec(), _d_spec(),
                  _w_spec(), _b_spec(), _w_spec()],
        out_specs=[_row_spec(), _col_spec()],
        out_shape=[jax.ShapeDtypeStruct((n, D), jnp.float32),
                   jax.ShapeDtypeStruct((n, 1), jnp.float32)],
    )(p, x, deg3, Wl0, bl0.reshape(1, D), Wr0)

    agg1 = _make_agg(D, False, ept)
    (q,) = agg1(h1, src, dst)

    wz = jnp.concatenate(
        [Wlo, Wro, jnp.zeros((D, D - 2 * Wlo.shape[1]), jnp.float32)], axis=1)
    bz = jnp.zeros((1, D), jnp.float32).at[0, 1].set(blo[0])
    z = pl.pallas_call(
        _c1_body,
        grid=grid,
        in_specs=[_p_spec(), _row_spec(), _col_spec(),
                  _w_spec(), _b_spec(), _w_spec(), _w_spec(), _b_spec()],
        out_specs=_row_spec(),
        out_shape=jax.ShapeDtypeStruct((n, D), jnp.float32),
    )(q, h1, rdeg, Wl1, bl1.reshape(1, D), Wr1, wz, bz)

    zl = z[:, 0]
    zrb = z[:, 1:2]
    agg2 = _make_agg(1, False, ept)
    (r,) = agg2(zl, src, dst)

    out = pl.pallas_call(
        _final_body,
        grid=grid,
        in_specs=[_d_spec(), _col_spec(), _col_spec()],
        out_specs=_col_spec(),
        out_shape=jax.ShapeDtypeStruct((n, 1), jnp.float32),
    )(r[:, :, None], rdeg, zrb)

    return out
